# baseline (speedup 1.0000x reference)
import jax
import jax.numpy as jnp
from jax.experimental import pallas as pl
from jax.experimental.pallas import tpu as pltpu

NUM_TOKENS = 16384
HIDDEN = 4096
NUM_EXPERTS = 64
TOP_K = 8
BLOCK_T = 1024


def _router_block(x_ref, wt_ref, merge_ref, map_ref, tpe_ref, logits_ref):
    logits = jnp.dot(x_ref[...], wt_ref[...],
                     preferred_element_type=jnp.float32)
    logits_ref[...] = logits

    neg_inf = jnp.float32(float("-inf"))

    masked = logits
    sel = jnp.zeros(logits.shape, dtype=jnp.bool_)
    for _ in range(TOP_K):
        m = jnp.max(masked, axis=1, keepdims=True)
        pick = masked == m
        sel = jnp.logical_or(sel, pick)
        masked = jnp.where(pick, neg_inf, masked)

    rowmax = jnp.max(logits, axis=1, keepdims=True)
    e = jnp.exp(logits - rowmax)
    e_sel = jnp.where(sel, e, 0.0)
    denom = jnp.sum(e_sel, axis=1, keepdims=True)
    merge_ref[...] = e_sel / denom

    sel_i32 = sel.astype(jnp.int32)
    map_ref[...] = sel_i32

    @pl.when(pl.program_id(0) == 0)
    def _init():
        tpe_ref[...] = jnp.zeros_like(tpe_ref)

    tpe_ref[...] += jnp.sum(sel_i32, axis=0, keepdims=True)


@jax.jit
def kernel(hidden_states, weight):
    wt = weight.T
    grid = NUM_TOKENS // BLOCK_T
    out_shapes = (
        jax.ShapeDtypeStruct((NUM_TOKENS, NUM_EXPERTS), jnp.float32),
        jax.ShapeDtypeStruct((NUM_TOKENS, NUM_EXPERTS), jnp.int32),
        jax.ShapeDtypeStruct((1, NUM_EXPERTS), jnp.int32),
        jax.ShapeDtypeStruct((NUM_TOKENS, NUM_EXPERTS), jnp.float32),
    )
    merging, routing_map, tpe, logits = pl.pallas_call(
        _router_block,
        grid=(grid,),
        in_specs=[
            pl.BlockSpec((BLOCK_T, HIDDEN), lambda i: (i, 0)),
            pl.BlockSpec((HIDDEN, NUM_EXPERTS), lambda i: (0, 0)),
        ],
        out_specs=(
            pl.BlockSpec((BLOCK_T, NUM_EXPERTS), lambda i: (i, 0)),
            pl.BlockSpec((BLOCK_T, NUM_EXPERTS), lambda i: (i, 0)),
            pl.BlockSpec((1, NUM_EXPERTS), lambda i: (0, 0)),
            pl.BlockSpec((BLOCK_T, NUM_EXPERTS), lambda i: (i, 0)),
        ),
        out_shape=out_shapes,
        compiler_params=pltpu.CompilerParams(
            dimension_semantics=("arbitrary",),
        ),
    )(hidden_states, wt)

    return (merging, routing_map, tpe.reshape(NUM_EXPERTS), logits)
